# baseline (speedup 1.0000x reference)
import functools

import jax
import jax.numpy as jnp
from jax import lax
from jax.experimental import pallas as pl
from jax.experimental.pallas import tpu as pltpu
from jax.experimental.pallas import tpu_sc as plsc

RADIUS = 1.0
SLOPE = 0.1
EPS = 1e-5

NC = 2
NS = 16
L = 16
NW = NC * NS

GB = 4
GBK = 2


def _leaky(x):
    return jnp.where(x >= 0, x, SLOPE * x)


def _bn(y, g, b):
    m = jnp.mean(y, axis=0, keepdims=True)
    v = jnp.mean((y - m) ** 2, axis=0, keepdims=True)
    return (y - m) * lax.rsqrt(v + EPS) * g + b


def _tc1_body(f_ref, w_ref, b_ref, g_ref, be_ref, o_ref):
    y = jnp.dot(f_ref[...], w_ref[...], preferred_element_type=jnp.float32)
    y = y + b_ref[...]
    o_ref[...] = _leaky(_bn(y, g_ref[...], be_ref[...]))


def _tc_stage1(features, W1, b1, g1, be1):
    n, _ = features.shape
    h = W1.shape[1]
    return pl.pallas_call(
        _tc1_body,
        out_shape=jax.ShapeDtypeStruct((n, h), jnp.float32),
        compiler_params=pltpu.CompilerParams(
            vmem_limit_bytes=100 * 1024 * 1024),
    )(features, W1, b1.reshape(1, -1), g1.reshape(1, -1), be1.reshape(1, -1))


def _rsqrt_approx(x):
    i = plsc.bitcast(x, jnp.int32)
    i = jnp.int32(0x5F3759DF) - lax.shift_right_logical(i, 1)
    y = plsc.bitcast(i, jnp.float32)
    for _ in range(2):
        y = y * (1.5 - 0.5 * x * y * y)
    return y


def _sc_fused_body(n_pad, k, h, m, c, chunk,
                   nb_ref, pts_ref, kp_ref, h1_ref, feat_ref, g_ref, mp_ref,
                   nb_v, own_v, kp_v, npts0, npts1, fn0, fn1, mr0, mr1,
                   infl_v, g_v, mp_v,
                   psem0, psem1, fsem0, fsem1, msem0, msem1):
    lb = 2 * L
    cj = c // lb
    hj = h // L
    mg_n = 3
    mg_sz = m // mg_n
    nbatch = chunk // GBK
    wid = lax.axis_index("s") * NC + lax.axis_index("c")
    base = wid * chunk
    pltpu.sync_copy(nb_ref.at[pl.ds(base * k, chunk * k)], nb_v)
    pltpu.sync_copy(pts_ref.at[pl.ds(base, chunk)], own_v)
    pltpu.sync_copy(kp_ref, kp_v)
    iot = lax.iota(jnp.int32, L)
    npts = (npts0, npts1)
    fn = (fn0, fn1)
    mr = (mr0, mr1)
    psems = (psem0, psem1)
    fsems = (fsem0, fsem1)
    msems = (msem0, msem1)

    def fire(i, b):
        idx = nb_v.at[pl.ds(i * GBK * k, GBK * k)]
        pltpu.async_copy(pts_ref.at[idx], npts[b], psems[b])
        pltpu.async_copy(h1_ref.at[idx], fn[b], fsems[b])
        pltpu.async_copy(feat_ref.at[idx], mr[b], msems[b])

    fire(0, 0)
    fire(1, 1)

    def batch2(i2, carry):
      for b in range(2):
        i = i2 * 2 + b
        npts_v = npts[b]
        fn_v = fn[b]
        pltpu.make_async_copy(
            pts_ref.at[pl.ds(0, GBK * k)], npts_v, psems[b]).wait()
        pltpu.make_async_copy(
            h1_ref.at[pl.ds(0, GBK * k)], fn_v, fsems[b]).wait()
        pltpu.make_async_copy(
            feat_ref.at[pl.ds(0, GBK * k)], mr[b], msems[b]).wait()
        for p in range(GBK):
            r0 = p * k

            def mpstep(kk, acc):
                return tuple(
                    jnp.maximum(acc[j], mr[b][r0 + kk, pl.ds(j * lb, lb)])
                    for j in range(cj))

            acc = lax.fori_loop(
                1, k, mpstep,
                tuple(mr[b][r0, pl.ds(j * lb, lb)] for j in range(cj)),
                unroll=4)
            for j in range(cj):
                mp_v[i * GBK + p, pl.ds(j * lb, lb)] = acc[j]
        for p in range(GBK):
            pt = i * GBK + p
            own = own_v[pt, pl.ds(0, L)]
            px = own[0]
            py = own[1]
            pz = own[2]
            rel = []
            for kh in range(k // L):
                row0 = p * k + kh * L
                rows = row0 + iot
                rel.append((
                    plsc.load_gather(npts_v, [rows, jnp.zeros((L,), jnp.int32)]) - px,
                    plsc.load_gather(npts_v, [rows, jnp.ones((L,), jnp.int32)]) - py,
                    plsc.load_gather(npts_v, [rows, jnp.full((L,), 2, jnp.int32)]) - pz,
                ))
            for mg0 in range(0, m, 5):
                res = []
                for mm in range(mg0, min(mg0 + 5, m)):
                    kpv = kp_v[mm, pl.ds(0, L)]
                    for kh in range(k // L):
                        rx, ry, rz = rel[kh]
                        dx = rx - kpv[0]
                        dy = ry - kpv[1]
                        dz = rz - kpv[2]
                        dd = dx * dx + dy * dy + dz * dz
                        d = dd * _rsqrt_approx(dd)
                        res.append((mm, kh, jnp.maximum(
                            0.0, 1.0 - d * (1.0 / RADIUS))))
                for mm, kh, val in res:
                    infl_v[mm, pl.ds(kh * L, L)] = val
            for mg in range(mg_n):
                def khstep(kh, acc):
                    iv = tuple(
                        infl_v[mg * mg_sz + mi, pl.ds(kh * L, L)]
                        for mi in range(mg_sz))

                    def kstep(kk, acc2):
                        row = p * k + kh * L + kk
                        f = tuple(fn_v[row, pl.ds(j * L, L)]
                                  for j in range(hj))
                        bidx = jnp.full((L,), kk, jnp.int32)
                        out = []
                        for mi in range(mg_sz):
                            s = iv[mi].at[bidx].get(
                                mode="promise_in_bounds")
                            out.append(tuple(acc2[mi][j] + s * f[j]
                                             for j in range(hj)))
                        return tuple(out)

                    return lax.fori_loop(0, L, kstep, acc, unroll=2)

                zero = jnp.zeros((L,), jnp.float32)
                acc0 = tuple(tuple(zero for _ in range(hj))
                             for _ in range(mg_sz))
                acc = lax.fori_loop(0, k // L, khstep, acc0)
                for mi in range(mg_sz):
                    for j in range(hj):
                        g_v[p, pl.ds((mg * mg_sz + mi) * h + j * L, L)] = acc[mi][j]
        pltpu.sync_copy(g_v, g_ref.at[pl.ds(base + i * GBK, GBK)])

        @pl.when(i + 2 < nbatch)
        def _():
            fire(i + 2, b)
      return carry

    lax.fori_loop(0, nbatch // 2, batch2, 0)
    pltpu.sync_copy(mp_v, mp_ref.at[pl.ds(base, chunk)])


def _sc_fused(pts_pad, nb_flat, kp_pad, h1, feats_bf, n_pad, k, chunk):
    h = h1.shape[1]
    m = kp_pad.shape[0]
    c = feats_bf.shape[1]
    mesh = plsc.VectorSubcoreMesh(
        core_axis_name="c", subcore_axis_name="s",
        num_cores=NC, num_subcores=NS)
    kfn = pl.kernel(
        functools.partial(_sc_fused_body, n_pad, k, h, m, c, chunk),
        out_type=[
            jax.ShapeDtypeStruct((n_pad, m * h), jnp.float32),
            jax.ShapeDtypeStruct((n_pad, c), jnp.bfloat16),
        ],
        mesh=mesh,
        scratch_types=[
            pltpu.VMEM((chunk * k,), jnp.int32),
            pltpu.VMEM((chunk, L), jnp.float32),
            pltpu.VMEM((m, L), jnp.float32),
            pltpu.VMEM((GBK * k, L), jnp.float32),
            pltpu.VMEM((GBK * k, L), jnp.float32),
            pltpu.VMEM((GBK * k, h), jnp.float32),
            pltpu.VMEM((GBK * k, h), jnp.float32),
            pltpu.VMEM((GBK * k, c), jnp.bfloat16),
            pltpu.VMEM((GBK * k, c), jnp.bfloat16),
            pltpu.VMEM((m, k), jnp.float32),
            pltpu.VMEM((GBK, m * h), jnp.float32),
            pltpu.VMEM((chunk, c), jnp.bfloat16),
            pltpu.SemaphoreType.DMA,
            pltpu.SemaphoreType.DMA,
            pltpu.SemaphoreType.DMA,
            pltpu.SemaphoreType.DMA,
            pltpu.SemaphoreType.DMA,
            pltpu.SemaphoreType.DMA,
        ],
        compiler_params=pltpu.CompilerParams(
            needs_layout_passes=False, use_tc_tiling_on_sc=False),
    )
    return kfn(nb_flat, pts_pad, kp_pad, h1, feats_bf)


def _tc2a_body(g_ref, wk_ref, mp_ref, ws_ref, bs_ref, y1_ref, y2_ref):
    y1_ref[...] = jnp.dot(g_ref[...], wk_ref[...],
                          preferred_element_type=jnp.float32)
    y2_ref[...] = jnp.dot(mp_ref[...].astype(jnp.float32), ws_ref[...],
                          preferred_element_type=jnp.float32) + bs_ref[...]


def _tc2b_body(y1_ref, g2_ref, be2_ref, w2_ref, b2_ref, y2_ref, gs_ref,
               bes_ref, o_ref):
    h2 = _leaky(_bn(y1_ref[...], g2_ref[...], be2_ref[...]))
    main = jnp.dot(h2, w2_ref[...], preferred_element_type=jnp.float32)
    main = main + b2_ref[...]
    sc = _bn(y2_ref[...], gs_ref[...], bes_ref[...])
    o_ref[...] = _leaky(main + sc)


def _tc_stage2(n, G, Wk, g2, be2, W2, b2, mp, Ws, bs, gs, bes):
    n_pad = G.shape[0]
    h = W2.shape[0]
    out_dim = W2.shape[1]
    wk_flat = Wk.reshape(-1, h)
    nb_rows = 10
    grid = (nb_rows,)
    blk = n_pad // nb_rows
    y1, y2 = pl.pallas_call(
        _tc2a_body,
        grid=grid,
        in_specs=[
            pl.BlockSpec((blk, wk_flat.shape[0]), lambda i: (i, 0)),
            pl.BlockSpec(wk_flat.shape, lambda i: (0, 0)),
            pl.BlockSpec((blk, Ws.shape[0]), lambda i: (i, 0)),
            pl.BlockSpec(Ws.shape, lambda i: (0, 0)),
            pl.BlockSpec((1, out_dim), lambda i: (0, 0)),
        ],
        out_specs=[
            pl.BlockSpec((blk, h), lambda i: (i, 0)),
            pl.BlockSpec((blk, out_dim), lambda i: (i, 0)),
        ],
        out_shape=[
            jax.ShapeDtypeStruct((n_pad, h), jnp.float32),
            jax.ShapeDtypeStruct((n_pad, out_dim), jnp.float32),
        ],
        compiler_params=pltpu.CompilerParams(
            vmem_limit_bytes=100 * 1024 * 1024),
    )(G, wk_flat, mp, Ws, bs.reshape(1, -1))
    return pl.pallas_call(
        _tc2b_body,
        grid=(1,),
        in_specs=[
            pl.BlockSpec((n, h), lambda i: (0, 0)),
            pl.BlockSpec((1, h), lambda i: (0, 0)),
            pl.BlockSpec((1, h), lambda i: (0, 0)),
            pl.BlockSpec((h, out_dim), lambda i: (0, 0)),
            pl.BlockSpec((1, out_dim), lambda i: (0, 0)),
            pl.BlockSpec((n, out_dim), lambda i: (0, 0)),
            pl.BlockSpec((1, out_dim), lambda i: (0, 0)),
            pl.BlockSpec((1, out_dim), lambda i: (0, 0)),
        ],
        out_specs=pl.BlockSpec((n, out_dim), lambda i: (0, 0)),
        out_shape=jax.ShapeDtypeStruct((n, out_dim), jnp.float32),
        compiler_params=pltpu.CompilerParams(
            vmem_limit_bytes=100 * 1024 * 1024),
    )(y1, g2.reshape(1, -1), be2.reshape(1, -1), W2, b2.reshape(1, -1),
      y2, gs.reshape(1, -1), bes.reshape(1, -1))


def kernel(points, features, neighbors, W1, b1, g1, be1, kp, Wk, g2, be2,
           W2, b2, Ws, bs, gs, bes):
    n, k = neighbors.shape
    m = kp.shape[0]
    h = W1.shape[1]

    chunk = -(-n // NW)
    chunk = -(-chunk // 8) * 8
    n_pad = chunk * NW

    nb_flat = jnp.pad(neighbors, ((0, n_pad - n), (0, 0))).reshape(-1)
    pts_pad = jnp.pad(points, ((0, n_pad - n), (0, L - points.shape[1])))
    kp_pad = jnp.pad(kp, ((0, 0), (0, L - kp.shape[1])))

    h1 = _tc_stage1(features, W1, b1, g1, be1)
    feats_bf = features.astype(jnp.bfloat16)
    G, mp = _sc_fused(pts_pad, nb_flat, kp_pad, h1, feats_bf, n_pad, k, chunk)
    return _tc_stage2(n, G, Wk, g2, be2, W2, b2, mp, Ws, bs, gs, bes)


# speedup vs baseline: 1.0414x; 1.0414x over previous
import functools

import jax
import jax.numpy as jnp
from jax import lax
from jax.experimental import pallas as pl
from jax.experimental.pallas import tpu as pltpu
from jax.experimental.pallas import tpu_sc as plsc

RADIUS = 1.0
SLOPE = 0.1
EPS = 1e-5

NC = 2
NS = 16
L = 16
NW = NC * NS

GB = 4
GBK = 2


def _leaky(x):
    return jnp.where(x >= 0, x, SLOPE * x)


def _bn(y, g, b):
    m = jnp.mean(y, axis=0, keepdims=True)
    v = jnp.mean((y - m) ** 2, axis=0, keepdims=True)
    return (y - m) * lax.rsqrt(v + EPS) * g + b


def _tc1_body(f_ref, w_ref, b_ref, g_ref, be_ref, o_ref):
    y = jnp.dot(f_ref[...], w_ref[...], preferred_element_type=jnp.float32)
    y = y + b_ref[...]
    o_ref[...] = _leaky(_bn(y, g_ref[...], be_ref[...]))


def _tc_stage1(features, W1, b1, g1, be1):
    n, _ = features.shape
    h = W1.shape[1]
    return pl.pallas_call(
        _tc1_body,
        out_shape=jax.ShapeDtypeStruct((n, h), jnp.float32),
        compiler_params=pltpu.CompilerParams(
            vmem_limit_bytes=100 * 1024 * 1024),
    )(features, W1, b1.reshape(1, -1), g1.reshape(1, -1), be1.reshape(1, -1))


def _rsqrt_approx(x):
    i = plsc.bitcast(x, jnp.int32)
    i = jnp.int32(0x5F3759DF) - lax.shift_right_logical(i, 1)
    y = plsc.bitcast(i, jnp.float32)
    for _ in range(2):
        y = y * (1.5 - 0.5 * x * y * y)
    return y


def _sc_fused_body(n_pad, k, h, m, c, chunk,
                   nb_ref, pts_ref, kp_ref, h1_ref, feat_ref, g_ref, mp_ref,
                   nb_v, own_v, kp_v, npts0, npts1, fn0, fn1, mr0, mr1,
                   infl_v, g_v, mp_v,
                   psem0, psem1, fsem0, fsem1, msem0, msem1):
    lb = 2 * L
    cj = c // lb
    hj = h // L
    mg_n = 3
    mg_sz = m // mg_n
    nbatch = chunk // GBK
    wid = lax.axis_index("s") * NC + lax.axis_index("c")
    base = wid * chunk
    pltpu.sync_copy(nb_ref.at[pl.ds(base * k, chunk * k)], nb_v)
    pltpu.sync_copy(pts_ref.at[pl.ds(base, chunk)], own_v)
    pltpu.sync_copy(kp_ref, kp_v)
    iot = lax.iota(jnp.int32, L)
    npts = (npts0, npts1)
    fn = (fn0, fn1)
    mr = (mr0, mr1)
    psems = (psem0, psem1)
    fsems = (fsem0, fsem1)
    msems = (msem0, msem1)

    def fire(i, b):
        idx = nb_v.at[pl.ds(i * GBK * k, GBK * k)]
        pltpu.async_copy(pts_ref.at[idx], npts[b], psems[b])
        pltpu.async_copy(h1_ref.at[idx], fn[b], fsems[b])
        pltpu.async_copy(feat_ref.at[idx], mr[b], msems[b])

    fire(0, 0)
    fire(1, 1)

    def batch2(i2, carry):
      for b in range(2):
        i = i2 * 2 + b
        npts_v = npts[b]
        fn_v = fn[b]
        pltpu.make_async_copy(
            pts_ref.at[pl.ds(0, GBK * k)], npts_v, psems[b]).wait()
        pltpu.make_async_copy(
            h1_ref.at[pl.ds(0, GBK * k)], fn_v, fsems[b]).wait()
        pltpu.make_async_copy(
            feat_ref.at[pl.ds(0, GBK * k)], mr[b], msems[b]).wait()
        for p in range(GBK):
            r0 = p * k

            def mpstep(kk, acc):
                return tuple(
                    jnp.maximum(acc[j], mr[b][r0 + kk, pl.ds(j * lb, lb)])
                    for j in range(cj))

            acc = lax.fori_loop(
                1, k, mpstep,
                tuple(mr[b][r0, pl.ds(j * lb, lb)] for j in range(cj)),
                unroll=4)
            for j in range(cj):
                mp_v[i * GBK + p, pl.ds(j * lb, lb)] = acc[j]
        for p in range(GBK):
            pt = i * GBK + p
            own = own_v[pt, pl.ds(0, L)]
            px = own[0]
            py = own[1]
            pz = own[2]
            rel = []
            for kh in range(k // L):
                row0 = p * k + kh * L
                rows = row0 + iot
                rel.append((
                    plsc.load_gather(npts_v, [rows, jnp.zeros((L,), jnp.int32)]) - px,
                    plsc.load_gather(npts_v, [rows, jnp.ones((L,), jnp.int32)]) - py,
                    plsc.load_gather(npts_v, [rows, jnp.full((L,), 2, jnp.int32)]) - pz,
                ))
            for mg0 in range(0, m, 5):
                res = []
                for mm in range(mg0, min(mg0 + 5, m)):
                    kpv = kp_v[mm, pl.ds(0, L)]
                    for kh in range(k // L):
                        rx, ry, rz = rel[kh]
                        dx = rx - kpv[0]
                        dy = ry - kpv[1]
                        dz = rz - kpv[2]
                        dd = dx * dx + dy * dy + dz * dz
                        d = dd * _rsqrt_approx(dd)
                        res.append((mm, kh, jnp.maximum(
                            0.0, 1.0 - d * (1.0 / RADIUS))))
                for mm, kh, val in res:
                    infl_v[mm, pl.ds(kh * L, L)] = val
            for mg in range(mg_n):
                def khstep(kh, acc):
                    iv = tuple(
                        infl_v[mg * mg_sz + mi, pl.ds(kh * L, L)]
                        for mi in range(mg_sz))

                    def kstep(kk, acc2):
                        row = p * k + kh * L + kk
                        f = tuple(fn_v[row, pl.ds(j * L, L)]
                                  for j in range(hj))
                        bidx = jnp.full((L,), kk, jnp.int32)
                        out = []
                        for mi in range(mg_sz):
                            s = iv[mi].at[bidx].get(
                                mode="promise_in_bounds")
                            out.append(tuple(acc2[mi][j] + s * f[j]
                                             for j in range(hj)))
                        return tuple(out)

                    return lax.fori_loop(0, L, kstep, acc)

                zero = jnp.zeros((L,), jnp.float32)
                acc0 = tuple(tuple(zero for _ in range(hj))
                             for _ in range(mg_sz))
                acc = lax.fori_loop(0, k // L, khstep, acc0)
                for mi in range(mg_sz):
                    for j in range(hj):
                        g_v[p, pl.ds((mg * mg_sz + mi) * h + j * L, L)] = acc[mi][j]
        pltpu.sync_copy(g_v, g_ref.at[pl.ds(base + i * GBK, GBK)])

        @pl.when(i + 2 < nbatch)
        def _():
            fire(i + 2, b)
      return carry

    lax.fori_loop(0, nbatch // 2, batch2, 0)
    pltpu.sync_copy(mp_v, mp_ref.at[pl.ds(base, chunk)])


def _sc_fused(pts_pad, nb_flat, kp_pad, h1, feats_bf, n_pad, k, chunk):
    h = h1.shape[1]
    m = kp_pad.shape[0]
    c = feats_bf.shape[1]
    mesh = plsc.VectorSubcoreMesh(
        core_axis_name="c", subcore_axis_name="s",
        num_cores=NC, num_subcores=NS)
    kfn = pl.kernel(
        functools.partial(_sc_fused_body, n_pad, k, h, m, c, chunk),
        out_type=[
            jax.ShapeDtypeStruct((n_pad, m * h), jnp.float32),
            jax.ShapeDtypeStruct((n_pad, c), jnp.bfloat16),
        ],
        mesh=mesh,
        scratch_types=[
            pltpu.VMEM((chunk * k,), jnp.int32),
            pltpu.VMEM((chunk, L), jnp.float32),
            pltpu.VMEM((m, L), jnp.float32),
            pltpu.VMEM((GBK * k, L), jnp.float32),
            pltpu.VMEM((GBK * k, L), jnp.float32),
            pltpu.VMEM((GBK * k, h), jnp.float32),
            pltpu.VMEM((GBK * k, h), jnp.float32),
            pltpu.VMEM((GBK * k, c), jnp.bfloat16),
            pltpu.VMEM((GBK * k, c), jnp.bfloat16),
            pltpu.VMEM((m, k), jnp.float32),
            pltpu.VMEM((GBK, m * h), jnp.float32),
            pltpu.VMEM((chunk, c), jnp.bfloat16),
            pltpu.SemaphoreType.DMA,
            pltpu.SemaphoreType.DMA,
            pltpu.SemaphoreType.DMA,
            pltpu.SemaphoreType.DMA,
            pltpu.SemaphoreType.DMA,
            pltpu.SemaphoreType.DMA,
        ],
        compiler_params=pltpu.CompilerParams(
            needs_layout_passes=False, use_tc_tiling_on_sc=False),
    )
    return kfn(nb_flat, pts_pad, kp_pad, h1, feats_bf)


def _tc2a_body(g_ref, wk_ref, mp_ref, ws_ref, bs_ref, y1_ref, y2_ref):
    y1_ref[...] = jnp.dot(g_ref[...], wk_ref[...],
                          preferred_element_type=jnp.float32)
    y2_ref[...] = jnp.dot(mp_ref[...].astype(jnp.float32), ws_ref[...],
                          preferred_element_type=jnp.float32) + bs_ref[...]


def _tc2b_body(y1_ref, g2_ref, be2_ref, w2_ref, b2_ref, y2_ref, gs_ref,
               bes_ref, o_ref):
    h2 = _leaky(_bn(y1_ref[...], g2_ref[...], be2_ref[...]))
    main = jnp.dot(h2, w2_ref[...], preferred_element_type=jnp.float32)
    main = main + b2_ref[...]
    sc = _bn(y2_ref[...], gs_ref[...], bes_ref[...])
    o_ref[...] = _leaky(main + sc)


def _tc_stage2(n, G, Wk, g2, be2, W2, b2, mp, Ws, bs, gs, bes):
    n_pad = G.shape[0]
    h = W2.shape[0]
    out_dim = W2.shape[1]
    wk_flat = Wk.reshape(-1, h)
    nb_rows = 10
    grid = (nb_rows,)
    blk = n_pad // nb_rows
    y1, y2 = pl.pallas_call(
        _tc2a_body,
        grid=grid,
        in_specs=[
            pl.BlockSpec((blk, wk_flat.shape[0]), lambda i: (i, 0)),
            pl.BlockSpec(wk_flat.shape, lambda i: (0, 0)),
            pl.BlockSpec((blk, Ws.shape[0]), lambda i: (i, 0)),
            pl.BlockSpec(Ws.shape, lambda i: (0, 0)),
            pl.BlockSpec((1, out_dim), lambda i: (0, 0)),
        ],
        out_specs=[
            pl.BlockSpec((blk, h), lambda i: (i, 0)),
            pl.BlockSpec((blk, out_dim), lambda i: (i, 0)),
        ],
        out_shape=[
            jax.ShapeDtypeStruct((n_pad, h), jnp.float32),
            jax.ShapeDtypeStruct((n_pad, out_dim), jnp.float32),
        ],
        compiler_params=pltpu.CompilerParams(
            vmem_limit_bytes=100 * 1024 * 1024),
    )(G, wk_flat, mp, Ws, bs.reshape(1, -1))
    return pl.pallas_call(
        _tc2b_body,
        grid=(1,),
        in_specs=[
            pl.BlockSpec((n, h), lambda i: (0, 0)),
            pl.BlockSpec((1, h), lambda i: (0, 0)),
            pl.BlockSpec((1, h), lambda i: (0, 0)),
            pl.BlockSpec((h, out_dim), lambda i: (0, 0)),
            pl.BlockSpec((1, out_dim), lambda i: (0, 0)),
            pl.BlockSpec((n, out_dim), lambda i: (0, 0)),
            pl.BlockSpec((1, out_dim), lambda i: (0, 0)),
            pl.BlockSpec((1, out_dim), lambda i: (0, 0)),
        ],
        out_specs=pl.BlockSpec((n, out_dim), lambda i: (0, 0)),
        out_shape=jax.ShapeDtypeStruct((n, out_dim), jnp.float32),
        compiler_params=pltpu.CompilerParams(
            vmem_limit_bytes=100 * 1024 * 1024),
    )(y1, g2.reshape(1, -1), be2.reshape(1, -1), W2, b2.reshape(1, -1),
      y2, gs.reshape(1, -1), bes.reshape(1, -1))


def kernel(points, features, neighbors, W1, b1, g1, be1, kp, Wk, g2, be2,
           W2, b2, Ws, bs, gs, bes):
    n, k = neighbors.shape
    m = kp.shape[0]
    h = W1.shape[1]

    chunk = -(-n // NW)
    chunk = -(-chunk // 8) * 8
    n_pad = chunk * NW

    nb_flat = jnp.pad(neighbors, ((0, n_pad - n), (0, 0))).reshape(-1)
    pts_pad = jnp.pad(points, ((0, n_pad - n), (0, L - points.shape[1])))
    kp_pad = jnp.pad(kp, ((0, 0), (0, L - kp.shape[1])))

    h1 = _tc_stage1(features, W1, b1, g1, be1)
    feats_bf = features.astype(jnp.bfloat16)
    G, mp = _sc_fused(pts_pad, nb_flat, kp_pad, h1, feats_bf, n_pad, k, chunk)
    return _tc_stage2(n, G, Wk, g2, be2, W2, b2, mp, Ws, bs, gs, bes)
